# Initial kernel scaffold; baseline (speedup 1.0000x reference)
import jax
import jax.numpy as jnp
from jax.experimental import pallas as pl


def kernel(x, edges, edge_attr, detector_labels, Wrel0, brel0, Wroot0, Wrel1, brel1, Wroot1, Wrel2, brel2, Wroot2, Wd0, bd0, Wd1, bd1, Wout, bout):
    raise NotImplementedError("write your pallas kernel here")


# speedup vs baseline: 1.9939x; 1.9939x over previous
import jax
import jax.numpy as jnp
from jax.experimental import pallas as pl

N_NODES = 10000
N_PAIRS = 80000
ROW_BLK = 1000
PAIR_BLK = 2000


def _layer_body(agg_ref, h_ref, wr_ref, wo_ref, br_ref, out_ref):
    acc = jnp.dot(agg_ref[...], wr_ref[...], preferred_element_type=jnp.float32)
    acc = acc + br_ref[...]
    acc = acc + jnp.dot(h_ref[...], wo_ref[...], preferred_element_type=jnp.float32)
    out_ref[...] = jnp.maximum(acc, 0.0)


def _layer(agg, h, Wr, br, Wo):
    cin, cout = h.shape[1], Wr.shape[0]
    return pl.pallas_call(
        _layer_body,
        grid=(N_NODES // ROW_BLK,),
        in_specs=[
            pl.BlockSpec((ROW_BLK, cin), lambda i: (i, 0)),
            pl.BlockSpec((ROW_BLK, cin), lambda i: (i, 0)),
            pl.BlockSpec((cin, cout), lambda i: (0, 0)),
            pl.BlockSpec((cin, cout), lambda i: (0, 0)),
            pl.BlockSpec((1, cout), lambda i: (0, 0)),
        ],
        out_specs=pl.BlockSpec((ROW_BLK, cout), lambda i: (i, 0)),
        out_shape=jax.ShapeDtypeStruct((N_NODES, cout), jnp.float32),
    )(agg, h, Wr.T, Wo.T, br[None, :])


def _mlp_body(hs_ref, hd_ref, eap_ref, w0_ref, bd0_ref, w1_ref, b1_ref,
              w2_ref, b2_ref, pv_ref, cv_ref):
    hs = hs_ref[...]
    hd = hd_ref[...]
    eap = eap_ref[...]

    def head(a):
        ef = jnp.concatenate([hs, a, hd], axis=1)
        z = jnp.dot(ef, w0_ref[...], preferred_element_type=jnp.float32)
        z = jnp.maximum(z + bd0_ref[...], 0.0)
        t = jnp.dot(z, w1_ref[...], preferred_element_type=jnp.float32)
        t = jnp.maximum(t + b1_ref[...], 0.0)
        f = jnp.dot(t, w2_ref[...], preferred_element_type=jnp.float32)
        return f + b2_ref[...]

    f0 = head(eap[:, 0:1])
    f1 = head(eap[:, 2:3])
    take1 = f1 < f0
    fm = jnp.where(take1, f1, f0)
    pv_ref[...] = 1.0 / (1.0 + jnp.exp(-fm))
    cv_ref[...] = jnp.where(take1, eap[:, 3:4], eap[:, 1:2])


def _mlp(hs, hd, eap, Wd0, bd0, Wd1, bd1, Wout, bout):
    return pl.pallas_call(
        _mlp_body,
        grid=(N_PAIRS // PAIR_BLK,),
        in_specs=[
            pl.BlockSpec((PAIR_BLK, 64), lambda i: (i, 0)),
            pl.BlockSpec((PAIR_BLK, 64), lambda i: (i, 0)),
            pl.BlockSpec((PAIR_BLK, 4), lambda i: (i, 0)),
            pl.BlockSpec((129, 64), lambda i: (0, 0)),
            pl.BlockSpec((1, 64), lambda i: (0, 0)),
            pl.BlockSpec((64, 32), lambda i: (0, 0)),
            pl.BlockSpec((1, 32), lambda i: (0, 0)),
            pl.BlockSpec((32, 1), lambda i: (0, 0)),
            pl.BlockSpec((1, 1), lambda i: (0, 0)),
        ],
        out_specs=[
            pl.BlockSpec((PAIR_BLK, 1), lambda i: (i, 0)),
            pl.BlockSpec((PAIR_BLK, 1), lambda i: (i, 0)),
        ],
        out_shape=[
            jax.ShapeDtypeStruct((N_PAIRS, 1), jnp.float32),
            jax.ShapeDtypeStruct((N_PAIRS, 1), jnp.float32),
        ],
    )(hs, hd, eap, Wd0.T, bd0[None, :], Wd1.T, bd1[None, :], Wout.T, bout[None, :])


def kernel(x, edges, edge_attr, detector_labels, Wrel0, brel0, Wroot0, Wrel1, brel1, Wroot1, Wrel2, brel2, Wroot2, Wd0, bd0, Wd1, bd1, Wout, bout):
    src, dst = edges[0], edges[1]
    w = edge_attr[:, 0] * edge_attr[:, 1]

    h = x
    for Wr, br, Wo in ((Wrel0, brel0, Wroot0), (Wrel1, brel1, Wroot1), (Wrel2, brel2, Wroot2)):
        msg = w[:, None] * h[src]
        agg = jax.ops.segment_sum(msg, dst, num_segments=N_NODES)
        h = _layer(agg, h, Wr, br, Wo)

    srcu = edges[0, ::2]
    dstu = edges[1, ::2]
    ea4 = edge_attr.reshape(-1, 4)
    key = srcu.astype(jnp.int32) * jnp.int32(N_NODES) + dstu.astype(jnp.int32)
    pi = jnp.argsort(key, stable=True)
    s = srcu[pi]
    d = dstu[pi]
    hs = h[s]
    hd = h[d]
    eap = ea4[pi]

    pv, cv = _mlp(hs, hd, eap, Wd0, bd0, Wd1, bd1, Wout, bout)
    e_out = jnp.stack([s, d], axis=0)
    return (e_out, pv[:, 0], cv[:, 0])


# speedup vs baseline: 4.3906x; 2.2021x over previous
import functools

import jax
import jax.numpy as jnp
from jax import lax
from jax.experimental import pallas as pl
from jax.experimental.pallas import tpu as pltpu
from jax.experimental.pallas import tpu_sc as plsc

N_NODES = 10000
N_ROWS_PAD = 10240
STRIPE = 640
N_PAIRS = 80000
NPAD = 81920
PER_TILE = NPAD // 32
CHUNK = 64
NCHUNK = PER_TILE // CHUNK
SCHUNK = 128
NSCHUNK = PER_TILE // SCHUNK
ROW_BLK = 1000
PAIR_BLK = 2000


def _make_segsum(C):
    mesh = plsc.VectorSubcoreMesh(core_axis_name="c", subcore_axis_name="s")

    @functools.partial(
        pl.kernel,
        mesh=mesh,
        out_type=jax.ShapeDtypeStruct((2, N_ROWS_PAD, C), jnp.float32),
        compiler_params=pltpu.CompilerParams(use_tc_tiling_on_sc=False),
        scratch_types=[
            pltpu.VMEM((PER_TILE,), jnp.int32),
            pltpu.VMEM((PER_TILE,), jnp.float32),
            pltpu.VMEM((PER_TILE,), jnp.float32),
            pltpu.VMEM((CHUNK, C), jnp.float32),
            pltpu.VMEM((CHUNK, C), jnp.float32),
            pltpu.VMEM((2 * CHUNK, C), jnp.float32),
            pltpu.VMEM((2 * CHUNK, C), jnp.float32),
            pltpu.VMEM((2 * CHUNK,), jnp.int32),
            pltpu.VMEM((2 * CHUNK,), jnp.int32),
            pltpu.VMEM((STRIPE, C), jnp.float32),
            pltpu.VMEM_SHARED((N_ROWS_PAD, C), jnp.float32),
        ] + [pltpu.SemaphoreType.DMA] * 3,
    )
    def segsum(srcs_hbm, dsts2_hbm, w0s_hbm, w1s_hbm, h_hbm, out_hbm,
               srcv, w0v, w1v, hbufA, hbufB, ubufA, ubufB,
               didxA, didxB, zbuf, accS,
               sH0, sH1, sS):
        cid = lax.axis_index("c")
        sid = lax.axis_index("s")
        wid = cid * 16 + sid
        base = wid * PER_TILE

        zeros16 = jnp.zeros((16,), jnp.float32)

        def zrow(j, _):
            for k in range(C // 16):
                zbuf[j, pl.ds(16 * k, 16)] = zeros16
            return 0

        lax.fori_loop(0, STRIPE, zrow, 0, unroll=4)
        pltpu.sync_copy(zbuf, accS.at[pl.ds(sid * STRIPE, STRIPE)])

        pltpu.sync_copy(srcs_hbm.at[pl.ds(base, PER_TILE)], srcv)
        pltpu.sync_copy(w0s_hbm.at[pl.ds(base, PER_TILE)], w0v)
        pltpu.sync_copy(w1s_hbm.at[pl.ds(base, PER_TILE)], w1v)

        plsc.subcore_barrier()

        hbufs = (hbufA, hbufB)
        ubufs = (ubufA, ubufB)
        didxs = (didxA, didxB)
        hsems = (sH0, sH1)

        def process(c, b):
            hbuf, ubuf, didx = hbufs[b], ubufs[b], didxs[b]
            pltpu.sync_copy(
                dsts2_hbm.at[pl.ds(2 * base + 2 * c * CHUNK, 2 * CHUNK)], didx)

            def group_body(g, _):
                g0 = c * CHUNK + 16 * g
                gv0 = w0v[pl.ds(g0, 16)]
                gv1 = w1v[pl.ds(g0, 16)]
                for l in range(16):
                    i = 16 * g + l
                    w0 = gv0[l]
                    w1 = gv1[l]
                    for k in range(C // 16):
                        hr = hbuf[i, pl.ds(16 * k, 16)]
                        ubuf[2 * i, pl.ds(16 * k, 16)] = w0 * hr
                        ubuf[2 * i + 1, pl.ds(16 * k, 16)] = w1 * hr
                return 0

            lax.fori_loop(0, CHUNK // 16, group_body, 0)

        def start_hgather(c, b):
            pltpu.async_copy(
                h_hbm.at[srcv.at[pl.ds(c * CHUNK, CHUNK)]], hbufs[b], hsems[b])

        def wait_hgather(b):
            pltpu.make_async_copy(
                h_hbm.at[pl.ds(0, CHUNK)], hbufs[b], hsems[b]).wait()

        def wait_scatter(b):
            pltpu.make_async_copy(
                h_hbm.at[pl.ds(0, 2 * CHUNK)], ubufs[b], sS).wait()

        start_hgather(0, 0)

        def chunk_pair(c2, _):
            for b in range(2):
                c = 2 * c2 + b

                @pl.when(c + 1 < NCHUNK)
                def _():
                    start_hgather(c + 1, 1 - b)

                wait_hgather(b)
                process(c, b)

                @pl.when(c >= 2)
                def _():
                    wait_scatter(b)

                pltpu.async_copy(ubufs[b], accS.at[didxs[b]], sS, add=True)
            return 0

        lax.fori_loop(0, NCHUNK // 2, chunk_pair, 0)
        wait_scatter(0)
        wait_scatter(1)

        plsc.subcore_barrier()
        pltpu.sync_copy(accS.at[pl.ds(sid * STRIPE, STRIPE)],
                        out_hbm.at[cid, pl.ds(sid * STRIPE, STRIPE)])

    return segsum


_segsum16 = _make_segsum(16)
_segsum32 = _make_segsum(32)


def _prep_body(ea_ref, srcu_ref, dstu_ref, w0_ref, w1_ref, key_ref):
    ea = ea_ref[...]
    w0_ref[...] = ea[:, 0:1] * ea[:, 1:2]
    w1_ref[...] = ea[:, 2:3] * ea[:, 3:4]
    key_ref[...] = srcu_ref[...] * jnp.int32(N_NODES) + dstu_ref[...]


def _prep(ea4, srcu, dstu):
    return pl.pallas_call(
        _prep_body,
        grid=(N_PAIRS // PAIR_BLK,),
        in_specs=[
            pl.BlockSpec((PAIR_BLK, 4), lambda i: (i, 0)),
            pl.BlockSpec((PAIR_BLK, 1), lambda i: (i, 0)),
            pl.BlockSpec((PAIR_BLK, 1), lambda i: (i, 0)),
        ],
        out_specs=[
            pl.BlockSpec((PAIR_BLK, 1), lambda i: (i, 0)),
            pl.BlockSpec((PAIR_BLK, 1), lambda i: (i, 0)),
            pl.BlockSpec((PAIR_BLK, 1), lambda i: (i, 0)),
        ],
        out_shape=[
            jax.ShapeDtypeStruct((N_PAIRS, 1), jnp.float32),
            jax.ShapeDtypeStruct((N_PAIRS, 1), jnp.float32),
            jax.ShapeDtypeStruct((N_PAIRS, 1), jnp.int32),
        ],
    )(ea4, srcu[:, None], dstu[:, None])


def _layer_body(agg_ref, h_ref, wr_ref, wo_ref, br_ref, out_ref):
    agg = agg_ref[0] + agg_ref[1]
    acc = jnp.dot(agg, wr_ref[...], preferred_element_type=jnp.float32)
    acc = acc + br_ref[...]
    acc = acc + jnp.dot(h_ref[...], wo_ref[...], preferred_element_type=jnp.float32)
    out_ref[...] = jnp.maximum(acc, 0.0)


def _layer(agg2, h, WrT, WoT, br):
    cin, cout = h.shape[1], WrT.shape[1]
    return pl.pallas_call(
        _layer_body,
        grid=(N_NODES // ROW_BLK,),
        in_specs=[
            pl.BlockSpec((2, ROW_BLK, cin), lambda i: (0, i, 0)),
            pl.BlockSpec((ROW_BLK, cin), lambda i: (i, 0)),
            pl.BlockSpec((cin, cout), lambda i: (0, 0)),
            pl.BlockSpec((cin, cout), lambda i: (0, 0)),
            pl.BlockSpec((1, cout), lambda i: (0, 0)),
        ],
        out_specs=pl.BlockSpec((ROW_BLK, cout), lambda i: (i, 0)),
        out_shape=jax.ShapeDtypeStruct((N_NODES, cout), jnp.float32),
    )(agg2, h, WrT, WoT, br[None, :])


def _mlp_body(hs_ref, hd_ref, eap_ref, w0_ref, bd0_ref, w1_ref, b1_ref,
              w2_ref, b2_ref, pv_ref, cv_ref):
    hs = hs_ref[...]
    hd = hd_ref[...]
    eap = eap_ref[...]

    def head(a):
        ef = jnp.concatenate([hs, a, hd], axis=1)
        z = jnp.dot(ef, w0_ref[...], preferred_element_type=jnp.float32)
        z = jnp.maximum(z + bd0_ref[...], 0.0)
        t = jnp.dot(z, w1_ref[...], preferred_element_type=jnp.float32)
        t = jnp.maximum(t + b1_ref[...], 0.0)
        f = jnp.dot(t, w2_ref[...], preferred_element_type=jnp.float32)
        return f + b2_ref[...]

    f0 = head(eap[:, 0:1])
    f1 = head(eap[:, 2:3])
    take1 = f1 < f0
    fm = jnp.where(take1, f1, f0)
    pv_ref[...] = 1.0 / (1.0 + jnp.exp(-fm))
    cv_ref[...] = jnp.where(take1, eap[:, 3:4], eap[:, 1:2])


def _mlp(hs, hd, eap, Wd0, bd0, Wd1, bd1, Wout, bout):
    return pl.pallas_call(
        _mlp_body,
        grid=(N_PAIRS // PAIR_BLK,),
        in_specs=[
            pl.BlockSpec((PAIR_BLK, 64), lambda i: (i, 0)),
            pl.BlockSpec((PAIR_BLK, 64), lambda i: (i, 0)),
            pl.BlockSpec((PAIR_BLK, 4), lambda i: (i, 0)),
            pl.BlockSpec((129, 64), lambda i: (0, 0)),
            pl.BlockSpec((1, 64), lambda i: (0, 0)),
            pl.BlockSpec((64, 32), lambda i: (0, 0)),
            pl.BlockSpec((1, 32), lambda i: (0, 0)),
            pl.BlockSpec((32, 1), lambda i: (0, 0)),
            pl.BlockSpec((1, 1), lambda i: (0, 0)),
        ],
        out_specs=[
            pl.BlockSpec((PAIR_BLK, 1), lambda i: (i, 0)),
            pl.BlockSpec((PAIR_BLK, 1), lambda i: (i, 0)),
        ],
        out_shape=[
            jax.ShapeDtypeStruct((N_PAIRS, 1), jnp.float32),
            jax.ShapeDtypeStruct((N_PAIRS, 1), jnp.float32),
        ],
    )(hs, hd, eap, Wd0.T, bd0[None, :], Wd1.T, bd1[None, :], Wout.T, bout[None, :])


def kernel(x, edges, edge_attr, detector_labels, Wrel0, brel0, Wroot0, Wrel1, brel1, Wroot1, Wrel2, brel2, Wroot2, Wd0, bd0, Wd1, bd1, Wout, bout):
    srcu = edges[0, ::2].astype(jnp.int32)
    dstu = edges[1, ::2].astype(jnp.int32)
    ea4 = edge_attr.reshape(-1, 4)

    w0, w1, key = _prep(ea4, srcu, dstu)
    w0, w1, key = w0[:, 0], w1[:, 0], key[:, 0]

    sigma = jnp.argsort(dstu, stable=True).astype(jnp.int32)
    zpad = jnp.zeros((NPAD - N_PAIRS,), jnp.float32)
    sigma_p = jnp.concatenate([sigma, jnp.arange(N_PAIRS, NPAD, dtype=jnp.int32)])
    srcu_p = jnp.concatenate([srcu, jnp.zeros((NPAD - N_PAIRS,), jnp.int32)])
    dstu_p = jnp.concatenate([dstu, jnp.zeros((NPAD - N_PAIRS,), jnp.int32)])
    w0_p = jnp.concatenate([w0, zpad])
    w1_p = jnp.concatenate([w1, zpad])

    x16 = jnp.pad(x, ((0, 0), (0, 11)))
    Wr0T = jnp.pad(Wrel0.T, ((0, 11), (0, 0)))
    Wo0T = jnp.pad(Wroot0.T, ((0, 11), (0, 0)))

    srcs = srcu_p[sigma_p]
    dsts2 = jnp.repeat(dstu_p[sigma_p], 2)
    w0s = w0_p[sigma_p]
    w1s = w1_p[sigma_p]

    agg2 = _segsum16(srcs, dsts2, w0s, w1s, x16)
    h = _layer(agg2, x16, Wr0T, Wo0T, brel0)
    agg2 = _segsum32(srcs, dsts2, w0s, w1s, h)
    h = _layer(agg2, h, Wrel1.T, Wroot1.T, brel1)
    agg2 = _segsum32(srcs, dsts2, w0s, w1s, h)
    h = _layer(agg2, h, Wrel2.T, Wroot2.T, brel2)

    pi = jnp.argsort(key, stable=True)
    s = srcu[pi]
    d = dstu[pi]
    hs = h[s]
    hd = h[d]
    eap = ea4[pi]

    pv, cv = _mlp(hs, hd, eap, Wd0, bd0, Wd1, bd1, Wout, bout)
    e_out = jnp.stack([s, d], axis=0).astype(edges.dtype)
    return (e_out, pv[:, 0], cv[:, 0])


# speedup vs baseline: 4.4923x; 1.0232x over previous
import functools

import jax
import jax.numpy as jnp
from jax import lax
from jax.experimental import pallas as pl
from jax.experimental.pallas import tpu as pltpu
from jax.experimental.pallas import tpu_sc as plsc

N_NODES = 10000
N_ROWS_PAD = 10240
STRIPE = 640
N_PAIRS = 80000
NPAD = 81920
PER_TILE = NPAD // 32
CHUNK = 64
NCHUNK = PER_TILE // CHUNK
SCHUNK = 128
NSCHUNK = PER_TILE // SCHUNK
ROW_BLK = 2000
PAIR_BLK = 8000


def _make_segsum(C):
    mesh = plsc.VectorSubcoreMesh(core_axis_name="c", subcore_axis_name="s")

    @functools.partial(
        pl.kernel,
        mesh=mesh,
        out_type=jax.ShapeDtypeStruct((2, N_ROWS_PAD, C), jnp.float32),
        compiler_params=pltpu.CompilerParams(use_tc_tiling_on_sc=False),
        scratch_types=[
            pltpu.VMEM((PER_TILE,), jnp.int32),
            pltpu.VMEM((PER_TILE,), jnp.float32),
            pltpu.VMEM((PER_TILE,), jnp.float32),
            pltpu.VMEM((CHUNK, C), jnp.float32),
            pltpu.VMEM((CHUNK, C), jnp.float32),
            pltpu.VMEM((2 * CHUNK, C), jnp.float32),
            pltpu.VMEM((2 * CHUNK, C), jnp.float32),
            pltpu.VMEM((2 * CHUNK,), jnp.int32),
            pltpu.VMEM((2 * CHUNK,), jnp.int32),
            pltpu.VMEM((STRIPE, C), jnp.float32),
            pltpu.VMEM_SHARED((N_ROWS_PAD, C), jnp.float32),
        ] + [pltpu.SemaphoreType.DMA] * 3,
    )
    def segsum(srcs_hbm, dsts2_hbm, w0s_hbm, w1s_hbm, h_hbm, out_hbm,
               srcv, w0v, w1v, hbufA, hbufB, ubufA, ubufB,
               didxA, didxB, zbuf, accS,
               sH0, sH1, sS):
        cid = lax.axis_index("c")
        sid = lax.axis_index("s")
        wid = cid * 16 + sid
        base = wid * PER_TILE

        zeros16 = jnp.zeros((16,), jnp.float32)

        def zrow(j, _):
            for k in range(C // 16):
                zbuf[j, pl.ds(16 * k, 16)] = zeros16
            return 0

        lax.fori_loop(0, STRIPE, zrow, 0, unroll=4)
        pltpu.sync_copy(zbuf, accS.at[pl.ds(sid * STRIPE, STRIPE)])

        pltpu.sync_copy(srcs_hbm.at[pl.ds(base, PER_TILE)], srcv)
        pltpu.sync_copy(w0s_hbm.at[pl.ds(base, PER_TILE)], w0v)
        pltpu.sync_copy(w1s_hbm.at[pl.ds(base, PER_TILE)], w1v)

        plsc.subcore_barrier()

        hbufs = (hbufA, hbufB)
        ubufs = (ubufA, ubufB)
        didxs = (didxA, didxB)
        hsems = (sH0, sH1)

        def process(c, b):
            hbuf, ubuf, didx = hbufs[b], ubufs[b], didxs[b]
            pltpu.sync_copy(
                dsts2_hbm.at[pl.ds(2 * base + 2 * c * CHUNK, 2 * CHUNK)], didx)

            def group_body(g, _):
                g0 = c * CHUNK + 16 * g
                gv0 = w0v[pl.ds(g0, 16)]
                gv1 = w1v[pl.ds(g0, 16)]
                for l in range(16):
                    i = 16 * g + l
                    w0 = gv0[l]
                    w1 = gv1[l]
                    for k in range(C // 16):
                        hr = hbuf[i, pl.ds(16 * k, 16)]
                        ubuf[2 * i, pl.ds(16 * k, 16)] = w0 * hr
                        ubuf[2 * i + 1, pl.ds(16 * k, 16)] = w1 * hr
                return 0

            lax.fori_loop(0, CHUNK // 16, group_body, 0)

        def start_hgather(c, b):
            pltpu.async_copy(
                h_hbm.at[srcv.at[pl.ds(c * CHUNK, CHUNK)]], hbufs[b], hsems[b])

        def wait_hgather(b):
            pltpu.make_async_copy(
                h_hbm.at[pl.ds(0, CHUNK)], hbufs[b], hsems[b]).wait()

        def wait_scatter(b):
            pltpu.make_async_copy(
                h_hbm.at[pl.ds(0, 2 * CHUNK)], ubufs[b], sS).wait()

        start_hgather(0, 0)

        def chunk_pair(c2, _):
            for b in range(2):
                c = 2 * c2 + b

                @pl.when(c + 1 < NCHUNK)
                def _():
                    start_hgather(c + 1, 1 - b)

                wait_hgather(b)
                process(c, b)

                @pl.when(c >= 2)
                def _():
                    wait_scatter(b)

                pltpu.async_copy(ubufs[b], accS.at[didxs[b]], sS, add=True)
            return 0

        lax.fori_loop(0, NCHUNK // 2, chunk_pair, 0)
        wait_scatter(0)
        wait_scatter(1)

        plsc.subcore_barrier()
        pltpu.sync_copy(accS.at[pl.ds(sid * STRIPE, STRIPE)],
                        out_hbm.at[cid, pl.ds(sid * STRIPE, STRIPE)])

    return segsum


_segsum16 = _make_segsum(16)
_segsum32 = _make_segsum(32)


def _prep_body(ea_ref, srcu_ref, dstu_ref, w0_ref, w1_ref, key_ref):
    ea = ea_ref[...]
    w0_ref[...] = ea[:, 0:1] * ea[:, 1:2]
    w1_ref[...] = ea[:, 2:3] * ea[:, 3:4]
    key_ref[...] = srcu_ref[...] * jnp.int32(N_NODES) + dstu_ref[...]


def _prep(ea4, srcu, dstu):
    return pl.pallas_call(
        _prep_body,
        grid=(N_PAIRS // PAIR_BLK,),
        in_specs=[
            pl.BlockSpec((PAIR_BLK, 4), lambda i: (i, 0)),
            pl.BlockSpec((PAIR_BLK, 1), lambda i: (i, 0)),
            pl.BlockSpec((PAIR_BLK, 1), lambda i: (i, 0)),
        ],
        out_specs=[
            pl.BlockSpec((PAIR_BLK, 1), lambda i: (i, 0)),
            pl.BlockSpec((PAIR_BLK, 1), lambda i: (i, 0)),
            pl.BlockSpec((PAIR_BLK, 1), lambda i: (i, 0)),
        ],
        out_shape=[
            jax.ShapeDtypeStruct((N_PAIRS, 1), jnp.float32),
            jax.ShapeDtypeStruct((N_PAIRS, 1), jnp.float32),
            jax.ShapeDtypeStruct((N_PAIRS, 1), jnp.int32),
        ],
    )(ea4, srcu[:, None], dstu[:, None])


def _layer_body(agg_ref, h_ref, wr_ref, wo_ref, br_ref, out_ref):
    agg = agg_ref[0] + agg_ref[1]
    acc = jnp.dot(agg, wr_ref[...], preferred_element_type=jnp.float32)
    acc = acc + br_ref[...]
    acc = acc + jnp.dot(h_ref[...], wo_ref[...], preferred_element_type=jnp.float32)
    out_ref[...] = jnp.maximum(acc, 0.0)


def _layer(agg2, h, WrT, WoT, br):
    cin, cout = h.shape[1], WrT.shape[1]
    return pl.pallas_call(
        _layer_body,
        grid=(N_NODES // ROW_BLK,),
        in_specs=[
            pl.BlockSpec((2, ROW_BLK, cin), lambda i: (0, i, 0)),
            pl.BlockSpec((ROW_BLK, cin), lambda i: (i, 0)),
            pl.BlockSpec((cin, cout), lambda i: (0, 0)),
            pl.BlockSpec((cin, cout), lambda i: (0, 0)),
            pl.BlockSpec((1, cout), lambda i: (0, 0)),
        ],
        out_specs=pl.BlockSpec((ROW_BLK, cout), lambda i: (i, 0)),
        out_shape=jax.ShapeDtypeStruct((N_NODES, cout), jnp.float32),
    )(agg2, h, WrT, WoT, br[None, :])


def _mlp_body(hs_ref, hd_ref, eap_ref, w0_ref, bd0_ref, w1_ref, b1_ref,
              w2_ref, b2_ref, pv_ref, cv_ref):
    hs = hs_ref[...]
    hd = hd_ref[...]
    eap = eap_ref[...]

    def head(a):
        ef = jnp.concatenate([hs, a, hd], axis=1)
        z = jnp.dot(ef, w0_ref[...], preferred_element_type=jnp.float32)
        z = jnp.maximum(z + bd0_ref[...], 0.0)
        t = jnp.dot(z, w1_ref[...], preferred_element_type=jnp.float32)
        t = jnp.maximum(t + b1_ref[...], 0.0)
        f = jnp.dot(t, w2_ref[...], preferred_element_type=jnp.float32)
        return f + b2_ref[...]

    f0 = head(eap[:, 0:1])
    f1 = head(eap[:, 2:3])
    take1 = f1 < f0
    fm = jnp.where(take1, f1, f0)
    pv_ref[...] = 1.0 / (1.0 + jnp.exp(-fm))
    cv_ref[...] = jnp.where(take1, eap[:, 3:4], eap[:, 1:2])


def _mlp(hs, hd, eap, Wd0, bd0, Wd1, bd1, Wout, bout):
    return pl.pallas_call(
        _mlp_body,
        grid=(N_PAIRS // PAIR_BLK,),
        in_specs=[
            pl.BlockSpec((PAIR_BLK, 64), lambda i: (i, 0)),
            pl.BlockSpec((PAIR_BLK, 64), lambda i: (i, 0)),
            pl.BlockSpec((PAIR_BLK, 4), lambda i: (i, 0)),
            pl.BlockSpec((129, 64), lambda i: (0, 0)),
            pl.BlockSpec((1, 64), lambda i: (0, 0)),
            pl.BlockSpec((64, 32), lambda i: (0, 0)),
            pl.BlockSpec((1, 32), lambda i: (0, 0)),
            pl.BlockSpec((32, 1), lambda i: (0, 0)),
            pl.BlockSpec((1, 1), lambda i: (0, 0)),
        ],
        out_specs=[
            pl.BlockSpec((PAIR_BLK, 1), lambda i: (i, 0)),
            pl.BlockSpec((PAIR_BLK, 1), lambda i: (i, 0)),
        ],
        out_shape=[
            jax.ShapeDtypeStruct((N_PAIRS, 1), jnp.float32),
            jax.ShapeDtypeStruct((N_PAIRS, 1), jnp.float32),
        ],
    )(hs, hd, eap, Wd0.T, bd0[None, :], Wd1.T, bd1[None, :], Wout.T, bout[None, :])


def kernel(x, edges, edge_attr, detector_labels, Wrel0, brel0, Wroot0, Wrel1, brel1, Wroot1, Wrel2, brel2, Wroot2, Wd0, bd0, Wd1, bd1, Wout, bout):
    srcu = edges[0, ::2].astype(jnp.int32)
    dstu = edges[1, ::2].astype(jnp.int32)
    ea4 = edge_attr.reshape(-1, 4)

    w0, w1, key = _prep(ea4, srcu, dstu)
    w0, w1, key = w0[:, 0], w1[:, 0], key[:, 0]

    sigma = jnp.argsort(dstu, stable=True).astype(jnp.int32)
    zpad = jnp.zeros((NPAD - N_PAIRS,), jnp.float32)
    sigma_p = jnp.concatenate([sigma, jnp.arange(N_PAIRS, NPAD, dtype=jnp.int32)])
    srcu_p = jnp.concatenate([srcu, jnp.zeros((NPAD - N_PAIRS,), jnp.int32)])
    dstu_p = jnp.concatenate([dstu, jnp.zeros((NPAD - N_PAIRS,), jnp.int32)])
    w0_p = jnp.concatenate([w0, zpad])
    w1_p = jnp.concatenate([w1, zpad])

    x16 = jnp.pad(x, ((0, 0), (0, 11)))
    Wr0T = jnp.pad(Wrel0.T, ((0, 11), (0, 0)))
    Wo0T = jnp.pad(Wroot0.T, ((0, 11), (0, 0)))

    srcs = srcu_p[sigma_p]
    dsts2 = jnp.repeat(dstu_p[sigma_p], 2)
    w0s = w0_p[sigma_p]
    w1s = w1_p[sigma_p]

    agg2 = _segsum16(srcs, dsts2, w0s, w1s, x16)
    h = _layer(agg2, x16, Wr0T, Wo0T, brel0)
    agg2 = _segsum32(srcs, dsts2, w0s, w1s, h)
    h = _layer(agg2, h, Wrel1.T, Wroot1.T, brel1)
    agg2 = _segsum32(srcs, dsts2, w0s, w1s, h)
    h = _layer(agg2, h, Wrel2.T, Wroot2.T, brel2)

    pi = jnp.argsort(key, stable=True)
    s = srcu[pi]
    d = dstu[pi]
    hs = h[s]
    hd = h[d]
    eap = ea4[pi]

    pv, cv = _mlp(hs, hd, eap, Wd0, bd0, Wd1, bd1, Wout, bout)
    e_out = jnp.stack([s, d], axis=0).astype(edges.dtype)
    return (e_out, pv[:, 0], cv[:, 0])
